# Initial kernel scaffold; baseline (speedup 1.0000x reference)
import jax
import jax.numpy as jnp
from jax.experimental import pallas as pl


def kernel(value, depth, pos, ve1, de1, se1, ve2, de2, se2, conv1_w, conv1_b, conv2_w, conv2_b):
    raise NotImplementedError("write your pallas kernel here")


# speedup vs baseline: 4.0976x; 4.0976x over previous
import jax
import jax.numpy as jnp
from jax import lax
from jax.experimental import pallas as pl
from jax.experimental.pallas import tpu as pltpu


def _make_row_kernel(T, T1, T2, W1, W2, OFF):
    NB = max(1, (T - 1).bit_length())

    def row_kernel(pk_ref, t1_ref, t2_ref, w1_ref, w2_ref, b1_ref, b2_ref, out_ref):
        A = pk_ref[0]
        iota_T = lax.broadcasted_iota(jnp.int32, (T, 1), 0)
        d = A[:, 1:2]
        maxd = jnp.max(d)
        idx = jnp.min(jnp.where(d == maxd, iota_T, T))
        v = A[:, 0:1]
        cnt = jnp.sum(jnp.where((iota_T >= idx) & (v != 0), 1, 0))

        iota1 = lax.broadcasted_iota(jnp.int32, (T1, 1), 0)
        A1 = jnp.where(iota1 < idx, A[:T1, :], 0)

        Ash = A
        for k in range(NB):
            sh = jnp.roll(Ash, -(1 << k), axis=0)
            bit = (idx >> k) & 1
            Ash = jnp.where(bit == 1, sh, Ash)
        iota2 = lax.broadcasted_iota(jnp.int32, (T2, 1), 0)
        A2 = jnp.where(iota2 < cnt, Ash[:T2, :], 0)

        iota_r = lax.broadcasted_iota(jnp.int32, (1, 256), 1)

        def embed(Ax, Tn, tab):
            O = jnp.zeros((Tn, 256), jnp.float32)
            for k in range(5):
                ik = Ax[:, k:k + 1] + OFF[k]
                O = O + jnp.where(ik == iota_r, 1.0, 0.0)
            return jnp.dot(O, tab, preferred_element_type=jnp.float32)

        x = embed(A1, T1, t1_ref[:, :])
        y = embed(A2, T2, t2_ref[:, :])

        iota_w2 = lax.broadcasted_iota(jnp.int32, (W2, 1), 0)
        iota_t2 = lax.broadcasted_iota(jnp.int32, (1, T2), 1)
        acc2 = jnp.zeros((W2, 32), jnp.float32) + b2_ref[:, :]
        for s in range(8):
            Q = jnp.where(iota_t2 == iota_w2 * 8 + s, 1.0, 0.0)
            ys = jnp.dot(Q, y, preferred_element_type=jnp.float32)
            acc2 = acc2 + jnp.dot(ys, w2_ref[s * 32:(s + 1) * 32, :],
                                  preferred_element_type=jnp.float32)

        mask2 = A1[:, 0:1] == 2
        mf = jnp.where(mask2, 1.0, 0.0)
        iota_c1 = lax.broadcasted_iota(jnp.int32, (1, T1), 1)
        Ltri = jnp.where(iota_c1 < iota1, 1.0, 0.0)
        pcum = jnp.dot(Ltri, mf, preferred_element_type=jnp.float32)
        pci = pcum.astype(jnp.int32)
        iota_w2r = lax.broadcasted_iota(jnp.int32, (1, W2), 1)
        OH = jnp.where(pci == iota_w2r, 1.0, 0.0)
        sub = jnp.dot(OH, acc2, preferred_element_type=jnp.float32)
        x = jnp.where(mask2, sub, x)

        iota_w1 = lax.broadcasted_iota(jnp.int32, (W1, 1), 0)
        iota_t1 = lax.broadcasted_iota(jnp.int32, (1, T1), 1)
        acc1 = jnp.zeros((W1, 256), jnp.float32) + b1_ref[:, :]
        for s in range(8):
            P = jnp.where(iota_t1 == iota_w1 * 8 + s, 1.0, 0.0)
            xs = jnp.dot(P, x, preferred_element_type=jnp.float32)
            acc1 = acc1 + jnp.dot(xs, w1_ref[s * 32:(s + 1) * 32, :],
                                  preferred_element_type=jnp.float32)
        out_ref[0] = acc1

    return row_kernel


def kernel(value, depth, pos, ve1, de1, se1, ve2, de2, se2, conv1_w, conv1_b, conv2_w, conv2_b):
    B, T = value.shape
    T1 = 512 + 32 * (B - 1)
    T2 = 4 * T1
    W1 = T1 // 8
    W2 = T2 // 8

    v32 = value.astype(jnp.int32)
    d32 = depth.astype(jnp.int32)
    p32 = pos.astype(jnp.int32)
    packed = jnp.concatenate([v32[:, :, None], d32[:, :, None], p32], axis=2)

    nv = ve1.shape[0]
    nd = de1.shape[0]
    ns = se1.shape[1]
    OFF = (0, nv, nv + nd, nv + nd + ns, nv + nd + 2 * ns)

    def pack_tab(ve, de, se):
        t = jnp.concatenate([ve, de, se[0], se[1], se[2]], axis=0)
        return jnp.pad(t, ((0, 256 - t.shape[0]), (0, 0)))

    tab1 = pack_tab(ve1, de1, se1)
    tab2 = pack_tab(ve2, de2, se2)
    w1r = jnp.transpose(conv1_w, (2, 1, 0)).reshape(8 * conv1_w.shape[1], conv1_w.shape[0])
    w2r = jnp.transpose(conv2_w, (2, 1, 0)).reshape(8 * conv2_w.shape[1], conv2_w.shape[0])
    b1 = conv1_b.reshape(1, -1)
    b2 = conv2_b.reshape(1, -1)

    row_kernel = _make_row_kernel(T, T1, T2, W1, W2, OFF)
    out = pl.pallas_call(
        row_kernel,
        grid=(B,),
        in_specs=[
            pl.BlockSpec((1, T, 5), lambda i: (i, 0, 0)),
            pl.BlockSpec((256, 32), lambda i: (0, 0)),
            pl.BlockSpec((256, 32), lambda i: (0, 0)),
            pl.BlockSpec((256, 256), lambda i: (0, 0)),
            pl.BlockSpec((256, 32), lambda i: (0, 0)),
            pl.BlockSpec((1, 256), lambda i: (0, 0)),
            pl.BlockSpec((1, 32), lambda i: (0, 0)),
        ],
        out_specs=pl.BlockSpec((1, W1, 256), lambda i: (i, 0, 0)),
        out_shape=jax.ShapeDtypeStruct((B, W1, 256), jnp.float32),
        compiler_params=pltpu.CompilerParams(
            dimension_semantics=("parallel",)),
    )(packed, tab1, tab2, w1r, w2r, b1, b2)
    return out
